# baseline (speedup 1.0000x reference)
import dataclasses
import functools

import jax
import jax.numpy as jnp
from jax import lax
from jax.experimental import pallas as pl
from jax.experimental.pallas import tpu as pltpu
from jax.experimental.pallas import tpu_sc as plsc

NC = 2
NS = 16
NW = NC * NS
PAD_D = 128
BLK = 128
PAD_COLS = 2048


def _pad_body(tt_ref, o_ref):
    o_ref[:, 0:64] = tt_ref[...].T


def _pad_table(table):
    v, d = table.shape
    return pl.pallas_call(
        _pad_body,
        grid=(pl.cdiv(v, PAD_COLS),),
        in_specs=[pl.BlockSpec((d, PAD_COLS), lambda i: (0, i))],
        out_specs=pl.BlockSpec((PAD_COLS, PAD_D), lambda i: (i, 0)),
        out_shape=jax.ShapeDtypeStruct((v, PAD_D), jnp.float32),
    )(table.T)


@jax.jit
def _embed_add(table_p, z_t, u_t):
    l, b = z_t.shape
    d = u_t.shape[1]
    blocks_per_w = b // (NW * BLK)
    mesh = plsc.VectorSubcoreMesh(core_axis_name="core", subcore_axis_name="sub")

    cp = pltpu.CompilerParams()
    if "needs_layout_passes" in pltpu.CompilerParams.__dataclass_fields__:
        cp = dataclasses.replace(cp, needs_layout_passes=False)

    @functools.partial(
        pl.kernel,
        out_type=jax.ShapeDtypeStruct((l, d, b), jnp.float32),
        mesh=mesh,
        compiler_params=cp,
        scratch_types=[
            pltpu.VMEM((l, BLK), jnp.int32),
            pltpu.VMEM((l, BLK), jnp.int32),
            pltpu.VMEM((BLK, PAD_D), jnp.float32),
            pltpu.VMEM((BLK, PAD_D), jnp.float32),
            pltpu.VMEM((d, BLK), jnp.float32),
            pltpu.VMEM((d, BLK), jnp.float32),
            pltpu.VMEM((d, BLK), jnp.float32),
            pltpu.VMEM((d, BLK), jnp.float32),
        ] + [pltpu.SemaphoreType.DMA] * 8,
    )
    def k(table_hbm, z_hbm, u_hbm, out_hbm,
          idx0, idx1, gb0, gb1, ub0, ub1, ob0, ob1,
          su0, su1, sg0, sg1, so0, so1, si0, si1):
        idxs = (idx0, idx1)
        gbs = (gb0, gb1)
        ubs = (ub0, ub1)
        obs = (ob0, ob1)
        sus = (su0, su1)
        sgs = (sg0, sg1)
        sos = (so0, so1)
        sis = (si0, si1)
        wid = lax.axis_index("sub") * NC + lax.axis_index("core")
        c0w = wid * blocks_per_w * BLK

        iota16 = lax.iota(jnp.int32, 16)

        def inc(jb):
            ib = idxs[jb]
            for r in range(l):
                for w in range(0, BLK, 16):
                    ib[r, pl.ds(w, 16)] = ib[r, pl.ds(w, 16)] + 1

        def idx_copy(blk, jb):
            col0 = pl.multiple_of(c0w + blk * BLK, BLK)
            return pltpu.make_async_copy(
                z_hbm.at[:, pl.ds(col0, BLK)], idxs[jb], sis[jb])

        def u_copy(blk, li, p):
            col0 = pl.multiple_of(c0w + blk * BLK, BLK)
            return pltpu.make_async_copy(
                u_hbm.at[li].at[:, pl.ds(col0, BLK)], ubs[p], sus[p])

        def g_copy(li, p, jb):
            return pltpu.make_async_copy(
                table_hbm.at[idxs[jb].at[li]], gbs[p], sgs[p])

        def o_copy(blk, li, p):
            col0 = pl.multiple_of(c0w + blk * BLK, BLK)
            return pltpu.make_async_copy(
                obs[p], out_hbm.at[li].at[:, pl.ds(col0, BLK)], sos[p])

        def start_a(blk, li, p, jb):
            u_copy(blk, li, p).start()
            g_copy(li, p, jb).start()

        def do_b(blk, li, p, drain_pred):
            u_copy(blk, li, p).wait()
            g_copy(0, p, 0).wait()

            @pl.when(drain_pred)
            def _():
                o_copy(blk, 0, p).wait()

            @pl.loop(0, d, step=8)
            def _(d0):
                for do in range(8):
                    dd = d0 + do
                    colv = jnp.full((16,), 1, jnp.int32) * dd
                    for w in range(0, BLK, 16):
                        gvals = plsc.load_gather(gbs[p], [iota16 + w, colv])
                        obs[p][dd, pl.ds(w, 16)] = (
                            ubs[p][dd, pl.ds(w, 16)] + gvals)

            o_copy(blk, li, p).start()

        c = idx_copy(0, 0)
        c.start()
        c.wait()
        inc(0)

        for blk in range(blocks_per_w):
            jb = blk % 2
            if blk + 1 < blocks_per_w:
                idx_copy(blk + 1, 1 - jb).start()

            start_a(blk, 0, 0, jb)

            @pl.loop(0, l // 2)
            def _(kk):
                li = kk * 2
                start_a(blk, li + 1, 1, jb)
                do_b(blk, li, 0, kk > 0)

                @pl.when(kk < l // 2 - 1)
                def _():
                    start_a(blk, li + 2, 0, jb)

                do_b(blk, li + 1, 1, kk > 0)

            o_copy(blk, 0, 0).wait()
            o_copy(blk, 0, 1).wait()

            if blk + 1 < blocks_per_w:
                idx_copy(blk + 1, 1 - jb).wait()
                inc(1 - jb)

    return k(table_p, z_t, u_t)


def kernel(z, u, table):
    table_p = _pad_table(table)
    z_t = z.astype(jnp.int32).T
    u_t = jnp.transpose(u, (1, 2, 0))
    out_t = _embed_add(table_p, z_t, u_t)
    v = jnp.transpose(out_t, (2, 0, 1))
    return (z, v)


# speedup vs baseline: 1.7895x; 1.7895x over previous
import dataclasses
import functools

import jax
import jax.numpy as jnp
from jax import lax
from jax.experimental import pallas as pl
from jax.experimental.pallas import tpu as pltpu
from jax.experimental.pallas import tpu_sc as plsc

NC = 2
NS = 16
NW = NC * NS
PAD_D = 128
BLK = 128
PAD_COLS = 2048


def _pad_body(tt_ref, o_ref):
    o_ref[:, 0:64] = tt_ref[...].T


def _pad_table(table):
    v, d = table.shape
    return pl.pallas_call(
        _pad_body,
        grid=(pl.cdiv(v, PAD_COLS),),
        in_specs=[pl.BlockSpec((d, PAD_COLS), lambda i: (0, i))],
        out_specs=pl.BlockSpec((PAD_COLS, PAD_D), lambda i: (i, 0)),
        out_shape=jax.ShapeDtypeStruct((v, PAD_D), jnp.float32),
    )(table.T)


@jax.jit
def _embed_add(table_p, z_t, u_t):
    l, b = z_t.shape
    d = u_t.shape[1]
    blocks_per_w = b // (NW * BLK)
    mesh = plsc.VectorSubcoreMesh(core_axis_name="core", subcore_axis_name="sub")

    cp = pltpu.CompilerParams()
    if "needs_layout_passes" in pltpu.CompilerParams.__dataclass_fields__:
        cp = dataclasses.replace(cp, needs_layout_passes=False)

    @functools.partial(
        pl.kernel,
        out_type=jax.ShapeDtypeStruct((l, d, b), jnp.float32),
        mesh=mesh,
        compiler_params=cp,
        scratch_types=[
            pltpu.VMEM((l, BLK), jnp.int32),
            pltpu.VMEM((l, BLK), jnp.int32),
            pltpu.VMEM((BLK, PAD_D), jnp.float32),
            pltpu.VMEM((BLK, PAD_D), jnp.float32),
            pltpu.VMEM((d, BLK), jnp.float32),
            pltpu.VMEM((d, BLK), jnp.float32),
            pltpu.VMEM((d, BLK), jnp.float32),
            pltpu.VMEM((d, BLK), jnp.float32),
        ] + [pltpu.SemaphoreType.DMA] * 8,
    )
    def k(table_hbm, z_hbm, u_hbm, out_hbm,
          idx0, idx1, gb0, gb1, ub0, ub1, ob0, ob1,
          su0, su1, sg0, sg1, so0, so1, si0, si1):
        idxs = (idx0, idx1)
        gbs = (gb0, gb1)
        ubs = (ub0, ub1)
        obs = (ob0, ob1)
        sus = (su0, su1)
        sgs = (sg0, sg1)
        sos = (so0, so1)
        sis = (si0, si1)
        wid = lax.axis_index("sub") * NC + lax.axis_index("core")
        c0w = wid * blocks_per_w * BLK

        iota16 = lax.iota(jnp.int32, 16)

        def inc(jb):
            ib = idxs[jb]
            for r in range(l):
                for w in range(0, BLK, 16):
                    ib[r, pl.ds(w, 16)] = ib[r, pl.ds(w, 16)] + 1

        def idx_copy(blk, jb):
            col0 = pl.multiple_of(c0w + blk * BLK, BLK)
            return pltpu.make_async_copy(
                z_hbm.at[:, pl.ds(col0, BLK)], idxs[jb], sis[jb])

        def u_copy(blk, li, p):
            col0 = pl.multiple_of(c0w + blk * BLK, BLK)
            return pltpu.make_async_copy(
                u_hbm.at[li].at[:, pl.ds(col0, BLK)], ubs[p], sus[p])

        def g_copy(li, p, jb):
            return pltpu.make_async_copy(
                table_hbm.at[idxs[jb].at[li]], gbs[p], sgs[p])

        def o_copy(blk, li, p):
            col0 = pl.multiple_of(c0w + blk * BLK, BLK)
            return pltpu.make_async_copy(
                obs[p], out_hbm.at[li].at[:, pl.ds(col0, BLK)], sos[p])

        def start_a(blk, li, p, jb):
            u_copy(blk, li, p).start()
            g_copy(li, p, jb).start()

        def do_b(blk, li, p, drain_pred):
            u_copy(blk, li, p).wait()
            g_copy(0, p, 0).wait()

            @pl.when(drain_pred)
            def _():
                o_copy(blk, 0, p).wait()

            @pl.loop(0, 16)
            def _(kp):
                perm = lax.rem(iota16 + kp, jnp.full((16,), 16, jnp.int32))
                for b0 in range(0, BLK, 16):
                    bvec = iota16 + b0
                    for d0 in range(0, d, 16):
                        dvec = perm + d0
                        gv = plsc.load_gather(gbs[p], [bvec, dvec])
                        uv = plsc.load_gather(ubs[p], [dvec, bvec])
                        plsc.store_scatter(obs[p], [dvec, bvec], gv + uv)

            o_copy(blk, li, p).start()

        c = idx_copy(0, 0)
        c.start()
        c.wait()
        inc(0)

        for blk in range(blocks_per_w):
            jb = blk % 2
            if blk + 1 < blocks_per_w:
                idx_copy(blk + 1, 1 - jb).start()

            start_a(blk, 0, 0, jb)

            @pl.loop(0, l // 2)
            def _(kk):
                li = kk * 2
                start_a(blk, li + 1, 1, jb)
                do_b(blk, li, 0, kk > 0)

                @pl.when(kk < l // 2 - 1)
                def _():
                    start_a(blk, li + 2, 0, jb)

                do_b(blk, li + 1, 1, kk > 0)

            o_copy(blk, 0, 0).wait()
            o_copy(blk, 0, 1).wait()

            if blk + 1 < blocks_per_w:
                idx_copy(blk + 1, 1 - jb).wait()
                inc(1 - jb)

    return k(table_p, z_t, u_t)


def kernel(z, u, table):
    table_p = _pad_table(table)
    z_t = z.astype(jnp.int32).T
    u_t = jnp.transpose(u, (1, 2, 0))
    out_t = _embed_add(table_p, z_t, u_t)
    v = jnp.transpose(out_t, (2, 0, 1))
    return (z, v)


# speedup vs baseline: 2.0337x; 1.1365x over previous
import dataclasses
import functools

import jax
import jax.numpy as jnp
from jax import lax
from jax.experimental import pallas as pl
from jax.experimental.pallas import tpu as pltpu
from jax.experimental.pallas import tpu_sc as plsc

NC = 2
NS = 16
NW = NC * NS
PAD_D = 128
BLK = 128
PAD_COLS = 4096


def _pad_body(tt_ref, o_ref):
    o_ref[:, 0:64] = tt_ref[...].T


def _pad_table(table):
    v, d = table.shape
    return pl.pallas_call(
        _pad_body,
        grid=(pl.cdiv(v, PAD_COLS),),
        in_specs=[pl.BlockSpec((d, PAD_COLS), lambda i: (0, i))],
        out_specs=pl.BlockSpec((PAD_COLS, PAD_D), lambda i: (i, 0)),
        out_shape=jax.ShapeDtypeStruct((v, PAD_D), jnp.float32),
    )(table.T)


@jax.jit
def _embed_add(table_p, z_t, u_t):
    l, b = z_t.shape
    d = u_t.shape[1]
    blocks_per_w = b // (NW * BLK)
    mesh = plsc.VectorSubcoreMesh(core_axis_name="core", subcore_axis_name="sub")

    cp = pltpu.CompilerParams()
    if "needs_layout_passes" in pltpu.CompilerParams.__dataclass_fields__:
        cp = dataclasses.replace(cp, needs_layout_passes=False)

    @functools.partial(
        pl.kernel,
        out_type=jax.ShapeDtypeStruct((l, d, b), jnp.float32),
        mesh=mesh,
        compiler_params=cp,
        scratch_types=[
            pltpu.VMEM((l, BLK), jnp.int32),
            pltpu.VMEM((l, BLK), jnp.int32),
            pltpu.VMEM((BLK, PAD_D), jnp.float32),
            pltpu.VMEM((BLK, PAD_D), jnp.float32),
            pltpu.VMEM((d, BLK), jnp.float32),
            pltpu.VMEM((d, BLK), jnp.float32),
            pltpu.VMEM((d, BLK), jnp.float32),
            pltpu.VMEM((d, BLK), jnp.float32),
        ] + [pltpu.SemaphoreType.DMA] * 8,
    )
    def k(table_hbm, z_hbm, u_hbm, out_hbm,
          idx0, idx1, gb0, gb1, ub0, ub1, ob0, ob1,
          su0, su1, sg0, sg1, so0, so1, si0, si1):
        idxs = (idx0, idx1)
        gbs = (gb0, gb1)
        ubs = (ub0, ub1)
        obs = (ob0, ob1)
        sus = (su0, su1)
        sgs = (sg0, sg1)
        sos = (so0, so1)
        sis = (si0, si1)
        wid = lax.axis_index("sub") * NC + lax.axis_index("core")
        c0w = wid * blocks_per_w * BLK

        iota16 = lax.iota(jnp.int32, 16)

        def inc(jb):
            ib = idxs[jb]
            for r in range(l):
                for w in range(0, BLK, 16):
                    ib[r, pl.ds(w, 16)] = ib[r, pl.ds(w, 16)] + 1

        def idx_copy(blk, jb):
            col0 = pl.multiple_of(c0w + blk * BLK, BLK)
            return pltpu.make_async_copy(
                z_hbm.at[:, pl.ds(col0, BLK)], idxs[jb], sis[jb])

        def u_copy(blk, li, p):
            col0 = pl.multiple_of(c0w + blk * BLK, BLK)
            return pltpu.make_async_copy(
                u_hbm.at[li].at[:, pl.ds(col0, BLK)], ubs[p], sus[p])

        def g_copy(li, p, jb):
            return pltpu.make_async_copy(
                table_hbm.at[idxs[jb].at[li]], gbs[p], sgs[p])

        def o_copy(blk, li, p):
            col0 = pl.multiple_of(c0w + blk * BLK, BLK)
            return pltpu.make_async_copy(
                obs[p], out_hbm.at[li].at[:, pl.ds(col0, BLK)], sos[p])

        def start_a(blk, li, p, jb):
            u_copy(blk, li, p).start()
            g_copy(li, p, jb).start()

        def do_b(blk, li, p, drain_pred):
            u_copy(blk, li, p).wait()
            g_copy(0, p, 0).wait()

            @pl.when(drain_pred)
            def _():
                o_copy(blk, 0, p).wait()

            @pl.loop(0, 16)
            def _(kp):
                perm = lax.rem(iota16 + kp, jnp.full((16,), 16, jnp.int32))
                for b0 in range(0, BLK, 16):
                    bvec = iota16 + b0
                    for d0 in range(0, d, 16):
                        dvec = perm + d0
                        gv = plsc.load_gather(gbs[p], [bvec, dvec])
                        uv = plsc.load_gather(ubs[p], [dvec, bvec])
                        plsc.store_scatter(obs[p], [dvec, bvec], gv + uv)

            o_copy(blk, li, p).start()

        c = idx_copy(0, 0)
        c.start()
        c.wait()
        inc(0)

        for blk in range(blocks_per_w):
            jb = blk % 2
            if blk + 1 < blocks_per_w:
                idx_copy(blk + 1, 1 - jb).start()

            start_a(blk, 0, 0, jb)

            @pl.loop(0, l // 2)
            def _(kk):
                li = kk * 2
                start_a(blk, li + 1, 1, jb)
                do_b(blk, li, 0, kk > 0)

                @pl.when(kk < l // 2 - 1)
                def _():
                    start_a(blk, li + 2, 0, jb)

                do_b(blk, li + 1, 1, kk > 0)

            o_copy(blk, 0, 0).wait()
            o_copy(blk, 0, 1).wait()

            if blk + 1 < blocks_per_w:
                idx_copy(blk + 1, 1 - jb).wait()
                inc(1 - jb)

    return k(table_p, z_t, u_t)


def kernel(z, u, table):
    table_p = _pad_table(table)
    z_t = z.astype(jnp.int32).T
    u_t = jnp.transpose(u, (1, 2, 0))
    out_t = _embed_add(table_p, z_t, u_t)
    v = jnp.transpose(out_t, (2, 0, 1))
    return (z, v)


# speedup vs baseline: 2.2098x; 1.0866x over previous
import dataclasses
import functools

import jax
import jax.numpy as jnp
from jax import lax
from jax.experimental import pallas as pl
from jax.experimental.pallas import tpu as pltpu
from jax.experimental.pallas import tpu_sc as plsc

NC = 2
NS = 16
NW = NC * NS
PAD_D = 128
BLK = 128
PAD_COLS = 8192


def _pad_body(tt_ref, o_ref):
    o_ref[:, 0:64] = tt_ref[...].T


def _pad_table(table):
    v, d = table.shape
    return pl.pallas_call(
        _pad_body,
        grid=(pl.cdiv(v, PAD_COLS),),
        in_specs=[pl.BlockSpec((d, PAD_COLS), lambda i: (0, i))],
        out_specs=pl.BlockSpec((PAD_COLS, PAD_D), lambda i: (i, 0)),
        out_shape=jax.ShapeDtypeStruct((v, PAD_D), jnp.float32),
    )(table.T)


@jax.jit
def _embed_add(table_p, z_t, u_t):
    l, b = z_t.shape
    d = u_t.shape[1]
    blocks_per_w = b // (NW * BLK)
    mesh = plsc.VectorSubcoreMesh(core_axis_name="core", subcore_axis_name="sub")

    cp = pltpu.CompilerParams()
    if "needs_layout_passes" in pltpu.CompilerParams.__dataclass_fields__:
        cp = dataclasses.replace(cp, needs_layout_passes=False)

    @functools.partial(
        pl.kernel,
        out_type=jax.ShapeDtypeStruct((l, d, b), jnp.float32),
        mesh=mesh,
        compiler_params=cp,
        scratch_types=[
            pltpu.VMEM((l, BLK), jnp.int32),
            pltpu.VMEM((l, BLK), jnp.int32),
            pltpu.VMEM((BLK, PAD_D), jnp.float32),
            pltpu.VMEM((BLK, PAD_D), jnp.float32),
            pltpu.VMEM((d, BLK), jnp.float32),
            pltpu.VMEM((d, BLK), jnp.float32),
            pltpu.VMEM((d, BLK), jnp.float32),
            pltpu.VMEM((d, BLK), jnp.float32),
        ] + [pltpu.SemaphoreType.DMA] * 8,
    )
    def k(table_hbm, z_hbm, u_hbm, out_hbm,
          idx0, idx1, gb0, gb1, ub0, ub1, ob0, ob1,
          su0, su1, sg0, sg1, so0, so1, si0, si1):
        idxs = (idx0, idx1)
        gbs = (gb0, gb1)
        ubs = (ub0, ub1)
        obs = (ob0, ob1)
        sus = (su0, su1)
        sgs = (sg0, sg1)
        sos = (so0, so1)
        sis = (si0, si1)
        wid = lax.axis_index("sub") * NC + lax.axis_index("core")
        c0w = wid * blocks_per_w * BLK

        iota16 = lax.iota(jnp.int32, 16)

        def inc(jb):
            ib = idxs[jb]
            for r in range(l):
                for w in range(0, BLK, 16):
                    ib[r, pl.ds(w, 16)] = ib[r, pl.ds(w, 16)] + 1

        def idx_copy(blk, jb):
            col0 = pl.multiple_of(c0w + blk * BLK, BLK)
            return pltpu.make_async_copy(
                z_hbm.at[:, pl.ds(col0, BLK)], idxs[jb], sis[jb])

        def u_copy(blk, li, p):
            col0 = pl.multiple_of(c0w + blk * BLK, BLK)
            return pltpu.make_async_copy(
                u_hbm.at[li].at[:, pl.ds(col0, BLK)], ubs[p], sus[p])

        def g_copy(li, p, jb):
            return pltpu.make_async_copy(
                table_hbm.at[idxs[jb].at[li]], gbs[p], sgs[p])

        def o_copy(blk, li, p):
            col0 = pl.multiple_of(c0w + blk * BLK, BLK)
            return pltpu.make_async_copy(
                obs[p], out_hbm.at[li].at[:, pl.ds(col0, BLK)], sos[p])

        def start_a(blk, li, p, jb):
            u_copy(blk, li, p).start()
            g_copy(li, p, jb).start()

        def do_b(blk, li, p, drain_pred):
            u_copy(blk, li, p).wait()
            g_copy(0, p, 0).wait()

            @pl.when(drain_pred)
            def _():
                o_copy(blk, 0, p).wait()

            @pl.loop(0, 16)
            def _(kp):
                perm = lax.rem(iota16 + kp, jnp.full((16,), 16, jnp.int32))
                for b0 in range(0, BLK, 16):
                    bvec = iota16 + b0
                    for d0 in range(0, d, 16):
                        dvec = perm + d0
                        gv = plsc.load_gather(gbs[p], [bvec, dvec])
                        uv = plsc.load_gather(ubs[p], [dvec, bvec])
                        plsc.store_scatter(obs[p], [dvec, bvec], gv + uv)

            o_copy(blk, li, p).start()

        c = idx_copy(0, 0)
        c.start()
        c.wait()
        inc(0)

        for blk in range(blocks_per_w):
            jb = blk % 2
            if blk + 1 < blocks_per_w:
                idx_copy(blk + 1, 1 - jb).start()

            start_a(blk, 0, 0, jb)

            @pl.loop(0, l // 2)
            def _(kk):
                li = kk * 2
                start_a(blk, li + 1, 1, jb)
                do_b(blk, li, 0, kk > 0)

                @pl.when(kk < l // 2 - 1)
                def _():
                    start_a(blk, li + 2, 0, jb)

                do_b(blk, li + 1, 1, kk > 0)

            o_copy(blk, 0, 0).wait()
            o_copy(blk, 0, 1).wait()

            if blk + 1 < blocks_per_w:
                idx_copy(blk + 1, 1 - jb).wait()
                inc(1 - jb)

    return k(table_p, z_t, u_t)


def kernel(z, u, table):
    table_p = _pad_table(table)
    z_t = z.astype(jnp.int32).T
    u_t = jnp.transpose(u, (1, 2, 0))
    out_t = _embed_add(table_p, z_t, u_t)
    v = jnp.transpose(out_t, (2, 0, 1))
    return (z, v)


# speedup vs baseline: 2.2612x; 1.0233x over previous
import dataclasses
import functools

import jax
import jax.numpy as jnp
from jax import lax
from jax.experimental import pallas as pl
from jax.experimental.pallas import tpu as pltpu
from jax.experimental.pallas import tpu_sc as plsc

NC = 2
NS = 16
NW = NC * NS
PAD_D = 128
BLK = 128
PAD_COLS = 16384


def _pad_body(tt_ref, o_ref):
    o_ref[:, 0:64] = tt_ref[...].T


def _pad_table(table):
    v, d = table.shape
    return pl.pallas_call(
        _pad_body,
        grid=(pl.cdiv(v, PAD_COLS),),
        in_specs=[pl.BlockSpec((d, PAD_COLS), lambda i: (0, i))],
        out_specs=pl.BlockSpec((PAD_COLS, PAD_D), lambda i: (i, 0)),
        out_shape=jax.ShapeDtypeStruct((v, PAD_D), jnp.float32),
    )(table.T)


@jax.jit
def _embed_add(table_p, z_t, u_t):
    l, b = z_t.shape
    d = u_t.shape[1]
    blocks_per_w = b // (NW * BLK)
    mesh = plsc.VectorSubcoreMesh(core_axis_name="core", subcore_axis_name="sub")

    cp = pltpu.CompilerParams()
    if "needs_layout_passes" in pltpu.CompilerParams.__dataclass_fields__:
        cp = dataclasses.replace(cp, needs_layout_passes=False)

    @functools.partial(
        pl.kernel,
        out_type=jax.ShapeDtypeStruct((l, d, b), jnp.float32),
        mesh=mesh,
        compiler_params=cp,
        scratch_types=[
            pltpu.VMEM((l, BLK), jnp.int32),
            pltpu.VMEM((l, BLK), jnp.int32),
            pltpu.VMEM((BLK, PAD_D), jnp.float32),
            pltpu.VMEM((BLK, PAD_D), jnp.float32),
            pltpu.VMEM((d, BLK), jnp.float32),
            pltpu.VMEM((d, BLK), jnp.float32),
            pltpu.VMEM((d, BLK), jnp.float32),
            pltpu.VMEM((d, BLK), jnp.float32),
        ] + [pltpu.SemaphoreType.DMA] * 8,
    )
    def k(table_hbm, z_hbm, u_hbm, out_hbm,
          idx0, idx1, gb0, gb1, ub0, ub1, ob0, ob1,
          su0, su1, sg0, sg1, so0, so1, si0, si1):
        idxs = (idx0, idx1)
        gbs = (gb0, gb1)
        ubs = (ub0, ub1)
        obs = (ob0, ob1)
        sus = (su0, su1)
        sgs = (sg0, sg1)
        sos = (so0, so1)
        sis = (si0, si1)
        wid = lax.axis_index("sub") * NC + lax.axis_index("core")
        c0w = wid * blocks_per_w * BLK

        iota16 = lax.iota(jnp.int32, 16)

        def inc(jb):
            ib = idxs[jb]
            for r in range(l):
                for w in range(0, BLK, 16):
                    ib[r, pl.ds(w, 16)] = ib[r, pl.ds(w, 16)] + 1

        def idx_copy(blk, jb):
            col0 = pl.multiple_of(c0w + blk * BLK, BLK)
            return pltpu.make_async_copy(
                z_hbm.at[:, pl.ds(col0, BLK)], idxs[jb], sis[jb])

        def u_copy(blk, li, p):
            col0 = pl.multiple_of(c0w + blk * BLK, BLK)
            return pltpu.make_async_copy(
                u_hbm.at[li].at[:, pl.ds(col0, BLK)], ubs[p], sus[p])

        def g_copy(li, p, jb):
            return pltpu.make_async_copy(
                table_hbm.at[idxs[jb].at[li]], gbs[p], sgs[p])

        def o_copy(blk, li, p):
            col0 = pl.multiple_of(c0w + blk * BLK, BLK)
            return pltpu.make_async_copy(
                obs[p], out_hbm.at[li].at[:, pl.ds(col0, BLK)], sos[p])

        def start_a(blk, li, p, jb):
            u_copy(blk, li, p).start()
            g_copy(li, p, jb).start()

        def do_b(blk, li, p, drain_pred):
            u_copy(blk, li, p).wait()
            g_copy(0, p, 0).wait()

            @pl.when(drain_pred)
            def _():
                o_copy(blk, 0, p).wait()

            @pl.loop(0, 16)
            def _(kp):
                perm = lax.rem(iota16 + kp, jnp.full((16,), 16, jnp.int32))
                for b0 in range(0, BLK, 16):
                    bvec = iota16 + b0
                    for d0 in range(0, d, 16):
                        dvec = perm + d0
                        gv = plsc.load_gather(gbs[p], [bvec, dvec])
                        uv = plsc.load_gather(ubs[p], [dvec, bvec])
                        plsc.store_scatter(obs[p], [dvec, bvec], gv + uv)

            o_copy(blk, li, p).start()

        c = idx_copy(0, 0)
        c.start()
        c.wait()
        inc(0)

        for blk in range(blocks_per_w):
            jb = blk % 2
            if blk + 1 < blocks_per_w:
                idx_copy(blk + 1, 1 - jb).start()

            start_a(blk, 0, 0, jb)

            @pl.loop(0, l // 2)
            def _(kk):
                li = kk * 2
                start_a(blk, li + 1, 1, jb)
                do_b(blk, li, 0, kk > 0)

                @pl.when(kk < l // 2 - 1)
                def _():
                    start_a(blk, li + 2, 0, jb)

                do_b(blk, li + 1, 1, kk > 0)

            o_copy(blk, 0, 0).wait()
            o_copy(blk, 0, 1).wait()

            if blk + 1 < blocks_per_w:
                idx_copy(blk + 1, 1 - jb).wait()
                inc(1 - jb)

    return k(table_p, z_t, u_t)


def kernel(z, u, table):
    table_p = _pad_table(table)
    z_t = z.astype(jnp.int32).T
    u_t = jnp.transpose(u, (1, 2, 0))
    out_t = _embed_add(table_p, z_t, u_t)
    v = jnp.transpose(out_t, (2, 0, 1))
    return (z, v)


# speedup vs baseline: 2.2766x; 1.0068x over previous
import dataclasses
import functools

import jax
import jax.numpy as jnp
from jax import lax
from jax.experimental import pallas as pl
from jax.experimental.pallas import tpu as pltpu
from jax.experimental.pallas import tpu_sc as plsc

NC = 2
NS = 16
NW = NC * NS
PAD_D = 128
BLK = 128
PAD_COLS = 32768


def _pad_body(tt_ref, o_ref):
    o_ref[:, 0:64] = tt_ref[...].T


def _pad_table(table):
    v, d = table.shape
    return pl.pallas_call(
        _pad_body,
        grid=(pl.cdiv(v, PAD_COLS),),
        in_specs=[pl.BlockSpec((d, PAD_COLS), lambda i: (0, i))],
        out_specs=pl.BlockSpec((PAD_COLS, PAD_D), lambda i: (i, 0)),
        out_shape=jax.ShapeDtypeStruct((v, PAD_D), jnp.float32),
    )(table.T)


@jax.jit
def _embed_add(table_p, z_t, u_t):
    l, b = z_t.shape
    d = u_t.shape[1]
    blocks_per_w = b // (NW * BLK)
    mesh = plsc.VectorSubcoreMesh(core_axis_name="core", subcore_axis_name="sub")

    cp = pltpu.CompilerParams()
    if "needs_layout_passes" in pltpu.CompilerParams.__dataclass_fields__:
        cp = dataclasses.replace(cp, needs_layout_passes=False)

    @functools.partial(
        pl.kernel,
        out_type=jax.ShapeDtypeStruct((l, d, b), jnp.float32),
        mesh=mesh,
        compiler_params=cp,
        scratch_types=[
            pltpu.VMEM((l, BLK), jnp.int32),
            pltpu.VMEM((l, BLK), jnp.int32),
            pltpu.VMEM((BLK, PAD_D), jnp.float32),
            pltpu.VMEM((BLK, PAD_D), jnp.float32),
            pltpu.VMEM((d, BLK), jnp.float32),
            pltpu.VMEM((d, BLK), jnp.float32),
            pltpu.VMEM((d, BLK), jnp.float32),
            pltpu.VMEM((d, BLK), jnp.float32),
        ] + [pltpu.SemaphoreType.DMA] * 8,
    )
    def k(table_hbm, z_hbm, u_hbm, out_hbm,
          idx0, idx1, gb0, gb1, ub0, ub1, ob0, ob1,
          su0, su1, sg0, sg1, so0, so1, si0, si1):
        idxs = (idx0, idx1)
        gbs = (gb0, gb1)
        ubs = (ub0, ub1)
        obs = (ob0, ob1)
        sus = (su0, su1)
        sgs = (sg0, sg1)
        sos = (so0, so1)
        sis = (si0, si1)
        wid = lax.axis_index("sub") * NC + lax.axis_index("core")
        c0w = wid * blocks_per_w * BLK

        iota16 = lax.iota(jnp.int32, 16)

        def inc(jb):
            ib = idxs[jb]
            for r in range(l):
                for w in range(0, BLK, 16):
                    ib[r, pl.ds(w, 16)] = ib[r, pl.ds(w, 16)] + 1

        def idx_copy(blk, jb):
            col0 = pl.multiple_of(c0w + blk * BLK, BLK)
            return pltpu.make_async_copy(
                z_hbm.at[:, pl.ds(col0, BLK)], idxs[jb], sis[jb])

        def u_copy(blk, li, p):
            col0 = pl.multiple_of(c0w + blk * BLK, BLK)
            return pltpu.make_async_copy(
                u_hbm.at[li].at[:, pl.ds(col0, BLK)], ubs[p], sus[p])

        def g_copy(li, p, jb):
            return pltpu.make_async_copy(
                table_hbm.at[idxs[jb].at[li]], gbs[p], sgs[p])

        def o_copy(blk, li, p):
            col0 = pl.multiple_of(c0w + blk * BLK, BLK)
            return pltpu.make_async_copy(
                obs[p], out_hbm.at[li].at[:, pl.ds(col0, BLK)], sos[p])

        def start_a(blk, li, p, jb):
            u_copy(blk, li, p).start()
            g_copy(li, p, jb).start()

        def do_b(blk, li, p, drain_pred):
            u_copy(blk, li, p).wait()
            g_copy(0, p, 0).wait()

            @pl.when(drain_pred)
            def _():
                o_copy(blk, 0, p).wait()

            @pl.loop(0, 16)
            def _(kp):
                perm = lax.rem(iota16 + kp, jnp.full((16,), 16, jnp.int32))
                for b0 in range(0, BLK, 16):
                    bvec = iota16 + b0
                    for d0 in range(0, d, 16):
                        dvec = perm + d0
                        gv = plsc.load_gather(gbs[p], [bvec, dvec])
                        uv = plsc.load_gather(ubs[p], [dvec, bvec])
                        plsc.store_scatter(obs[p], [dvec, bvec], gv + uv)

            o_copy(blk, li, p).start()

        c = idx_copy(0, 0)
        c.start()
        c.wait()
        inc(0)

        for blk in range(blocks_per_w):
            jb = blk % 2
            if blk + 1 < blocks_per_w:
                idx_copy(blk + 1, 1 - jb).start()

            start_a(blk, 0, 0, jb)

            @pl.loop(0, l // 2)
            def _(kk):
                li = kk * 2
                start_a(blk, li + 1, 1, jb)
                do_b(blk, li, 0, kk > 0)

                @pl.when(kk < l // 2 - 1)
                def _():
                    start_a(blk, li + 2, 0, jb)

                do_b(blk, li + 1, 1, kk > 0)

            o_copy(blk, 0, 0).wait()
            o_copy(blk, 0, 1).wait()

            if blk + 1 < blocks_per_w:
                idx_copy(blk + 1, 1 - jb).wait()
                inc(1 - jb)

    return k(table_p, z_t, u_t)


def kernel(z, u, table):
    table_p = _pad_table(table)
    z_t = z.astype(jnp.int32).T
    u_t = jnp.transpose(u, (1, 2, 0))
    out_t = _embed_add(table_p, z_t, u_t)
    v = jnp.transpose(out_t, (2, 0, 1))
    return (z, v)
